# baseline (speedup 1.0000x reference)
import functools

import jax
import jax.numpy as jnp
from jax import lax
from jax.experimental import pallas as pl
from jax.experimental.pallas import tpu as pltpu
from jax.experimental.pallas import tpu_sc as plsc

B = 4
S = 8192
D = 768
L = 16
NC = 2
NS = 16
NW = NC * NS
ROWS_PER_W = S // NW
CHUNK = 16
N_CHUNKS = ROWS_PER_W // CHUNK
CGROUPS = D // L


def _pe_add(x_hbm, enc_hbm, out_hbm, *scr):
    enc_v = [scr[0], scr[1]]
    xv = [scr[2], scr[3]]
    esem = [scr[4], scr[5]]
    isem = [scr[6], scr[7]]
    osem = [scr[8], scr[9]]
    wid = lax.axis_index("s") * NC + lax.axis_index("c")
    base = wid * ROWS_PER_W

    def start_enc(ci, p):
        pltpu.async_copy(
            enc_hbm.at[pl.ds(base + ci * CHUNK, CHUNK)], enc_v[p], esem[p])

    def wait_enc(p):
        pltpu.make_async_copy(
            enc_hbm.at[pl.ds(0, CHUNK)], enc_v[p], esem[p]).wait()

    def start_in(ci, p):
        pltpu.async_copy(
            x_hbm.at[:, pl.ds(base + ci * CHUNK, CHUNK)], xv[p], isem[p])

    def wait_in(p):
        pltpu.make_async_copy(
            x_hbm.at[:, pl.ds(0, CHUNK)], xv[p], isem[p]).wait()

    def wait_out(p):
        pltpu.make_async_copy(
            xv[p], out_hbm.at[:, pl.ds(0, CHUNK)], osem[p]).wait()

    def add_chunk(buf, ev):
        for b in range(B):
            @plsc.parallel_loop(0, CHUNK, 1, unroll=2)
            def _(r, b=b):
                for c in range(CGROUPS):
                    sl = pl.ds(c * L, L)
                    plsc.addupdate(buf.at[b, r, sl], ev[r, sl])

    def phase(ci, p):
        s0 = base + ci * CHUNK
        ci_next = jnp.minimum(ci + 1, N_CHUNKS - 1)
        start_enc(ci_next, 1 - p)

        @pl.when(ci > 0)
        def _():
            wait_out(1 - p)

        start_in(ci_next, 1 - p)
        wait_enc(p)
        wait_in(p)
        add_chunk(xv[p], enc_v[p])
        pltpu.async_copy(xv[p], out_hbm.at[:, pl.ds(s0, CHUNK)], osem[p])

    start_enc(0, 0)
    start_in(0, 0)

    def pair_body(ci2, carry):
        phase(2 * ci2, 0)
        phase(2 * ci2 + 1, 1)
        return carry

    lax.fori_loop(0, N_CHUNKS // 2, pair_body, 0)
    wait_enc(0)
    wait_out(1)
    wait_in(0)


@jax.jit
def kernel(x, encoding):
    mesh = plsc.VectorSubcoreMesh(core_axis_name="c", subcore_axis_name="s")
    scratch = [pltpu.VMEM((CHUNK, D), jnp.float32)] * 2
    scratch += [pltpu.VMEM((B, CHUNK, D), jnp.float32)] * 2
    scratch += [pltpu.SemaphoreType.DMA] * 6
    f = functools.partial(
        pl.kernel,
        mesh=mesh,
        out_type=jax.ShapeDtypeStruct((B, S, D), jnp.float32),
        scratch_types=scratch,
    )(_pe_add)
    return f(x, encoding)


# speedup vs baseline: 1.1884x; 1.1884x over previous
import functools

import jax
import jax.numpy as jnp
from jax import lax
from jax.experimental import pallas as pl
from jax.experimental.pallas import tpu as pltpu
from jax.experimental.pallas import tpu_sc as plsc

B = 4
NP = 2
S = 8192
D = 768
L = 16
NC = 2
NS = 16
NW = NC * NS
ROWS_PER_W = S // NW
CHUNK = 16
N_CHUNKS = ROWS_PER_W // CHUNK
CGROUPS = D // L


def _pe_add(x_hbm, enc_hbm, out_hbm, *scr):
    enc_v = [scr[0], scr[1]]
    xv = [[scr[2], scr[3]], [scr[4], scr[5]]]
    esem = [scr[6], scr[7]]
    isem = [[scr[8], scr[9]], [scr[10], scr[11]]]
    osem = [[scr[12], scr[13]], [scr[14], scr[15]]]
    wid = lax.axis_index("s") * NC + lax.axis_index("c")
    base = wid * ROWS_PER_W

    def start_enc(ci, p):
        pltpu.async_copy(
            enc_hbm.at[pl.ds(base + ci * CHUNK, CHUNK)], enc_v[p], esem[p])

    def wait_enc(p):
        pltpu.make_async_copy(
            enc_hbm.at[pl.ds(0, CHUNK)], enc_v[p], esem[p]).wait()

    def start_in(ci, g, p):
        pltpu.async_copy(
            x_hbm.at[pl.ds(2 * g, 2), pl.ds(base + ci * CHUNK, CHUNK)],
            xv[g][p], isem[g][p])

    def wait_in(g, p):
        pltpu.make_async_copy(
            x_hbm.at[pl.ds(0, 2), pl.ds(0, CHUNK)], xv[g][p],
            isem[g][p]).wait()

    def wait_out(g, p):
        pltpu.make_async_copy(
            xv[g][p], out_hbm.at[pl.ds(0, 2), pl.ds(0, CHUNK)],
            osem[g][p]).wait()

    def add_chunk(buf, ev):
        for b in range(2):
            @plsc.parallel_loop(0, CHUNK, 1, unroll=2)
            def _(r, b=b):
                for c in range(CGROUPS):
                    sl = pl.ds(c * L, L)
                    plsc.addupdate(buf.at[b, r, sl], ev[r, sl])

    def phase(ci, p):
        s0 = base + ci * CHUNK
        ci_next = jnp.minimum(ci + 1, N_CHUNKS - 1)
        start_enc(ci_next, 1 - p)
        for g in range(NP):
            @pl.when(ci > 0)
            def _():
                wait_out(g, 1 - p)

            start_in(ci_next, g, 1 - p)
        wait_enc(p)
        for g in range(NP):
            wait_in(g, p)
            add_chunk(xv[g][p], enc_v[p])
            pltpu.async_copy(
                xv[g][p],
                out_hbm.at[pl.ds(2 * g, 2), pl.ds(s0, CHUNK)], osem[g][p])

    start_enc(0, 0)
    for g in range(NP):
        start_in(0, g, 0)

    def pair_body(ci2, carry):
        phase(2 * ci2, 0)
        phase(2 * ci2 + 1, 1)
        return carry

    lax.fori_loop(0, N_CHUNKS // 2, pair_body, 0)
    wait_enc(0)
    for g in range(NP):
        wait_out(g, 1)
        wait_in(g, 0)


@jax.jit
def kernel(x, encoding):
    mesh = plsc.VectorSubcoreMesh(core_axis_name="c", subcore_axis_name="s")
    scratch = [pltpu.VMEM((CHUNK, D), jnp.float32)] * 2
    scratch += [pltpu.VMEM((2, CHUNK, D), jnp.float32)] * 4
    scratch += [pltpu.SemaphoreType.DMA] * 10
    f = functools.partial(
        pl.kernel,
        mesh=mesh,
        out_type=jax.ShapeDtypeStruct((B, S, D), jnp.float32),
        scratch_types=scratch,
    )(_pe_add)
    return f(x, encoding)
